# baseline (speedup 1.0000x reference)
import functools

import jax
import jax.numpy as jnp
from jax import lax
from jax.experimental import pallas as pl
from jax.experimental.pallas import tpu as pltpu
from jax.experimental.pallas import tpu_sc as plsc

IN_DIM = 4096
OUT_DIM = 4096
BATCH = 256
PKW = BATCH // 2

NUM_TILES = 16
NW = 2 * NUM_TILES
CHUNK = 80
LANES = 16


def _sc_spmm(nchunk):
  mesh = plsc.VectorSubcoreMesh(core_axis_name="c", subcore_axis_name="s")

  @functools.partial(
      pl.kernel,
      mesh=mesh,
      out_type=jax.ShapeDtypeStruct((2, OUT_DIM, 2, 128), jnp.float32),
      scratch_types=[
          pltpu.VMEM((nchunk, CHUNK), jnp.int32),
          pltpu.VMEM((nchunk, CHUNK), jnp.float32),
          pltpu.VMEM((2, CHUNK), jnp.int32),
          pltpu.VMEM((2, CHUNK), jnp.int32),
          pltpu.VMEM((CHUNK, PKW), jnp.int32),
          pltpu.VMEM((CHUNK, PKW), jnp.int32),
          pltpu.VMEM((CHUNK, 2, 128), jnp.float32),
          pltpu.VMEM_SHARED((OUT_DIM, 2, 128), jnp.float32),
          pltpu.SemaphoreType.DMA,
          pltpu.SemaphoreType.DMA,
      ],
  )
  def k(xi_hbm, rc_hbm, val_hbm, out_hbm,
        rc_v, val_v, colv, rowv, gbufa, gbufb, sbuf, acc,
        sema, semb):
    cid = lax.axis_index("c")
    sid = lax.axis_index("s")
    wid = cid * NUM_TILES + sid

    pltpu.sync_copy(rc_hbm.at[wid], rc_v)
    pltpu.sync_copy(val_hbm.at[wid], val_v)

    def _unpack(ch, bi):
      def _u(g, _):
        s = pl.ds(g * LANES, LANES)
        rc = rc_v[ch, s]
        colv[bi, s] = rc & 4095
        rowv[bi, s] = lax.shift_right_logical(rc, 12)
        return 0
      lax.fori_loop(0, CHUNK // LANES, _u, 0, unroll=True)

    def _zrow(i, _):
      def _zlane(g, _):
        h, off = divmod(0, 1)
        return 0
      for hh in range(2):
        def _zl(g, _, hh=hh):
          sbuf[i, hh, pl.ds(g * LANES, LANES)] = jnp.zeros(
              (LANES,), jnp.float32)
          return 0
        lax.fori_loop(0, 128 // LANES, _zl, 0, unroll=True)
      return 0
    lax.fori_loop(0, CHUNK, _zrow, 0)
    rows_per_tile = OUT_DIM // NUM_TILES
    nz = -(-rows_per_tile // CHUNK)
    for zi in range(nz):
      zbase = min(zi * CHUNK, rows_per_tile - CHUNK)
      pltpu.sync_copy(sbuf, acc.at[pl.ds(sid * rows_per_tile + zbase, CHUNK)])
    plsc.subcore_barrier()

    def _bcast(vvec, l):
      return lax.gather(
          vvec,
          jnp.full((LANES, 1), l, jnp.int32),
          lax.GatherDimensionNumbers(
              offset_dims=(), collapsed_slice_dims=(0,),
              start_index_map=(0,)),
          (1,),
          mode=lax.GatherScatterMode.PROMISE_IN_BOUNDS)

    himask = jnp.full((LANES,), -65536, jnp.int32)

    def _scale(gbuf, vrow):
      def _s16(j16, _):
        vvec = val_v[vrow, pl.ds(j16 * LANES, LANES)]
        for l in range(LANES):
          v = _bcast(vvec, l)
          j = j16 * LANES + l
          for g in range(PKW // LANES):
            s = pl.ds(g * LANES, LANES)
            u = gbuf[j, s]
            flo = lax.bitcast_convert_type(lax.shift_left(u, 16), jnp.float32)
            fhi = lax.bitcast_convert_type(u & himask, jnp.float32)
            c0 = g * 2 * LANES
            sbuf[j, c0 // 128, pl.ds(c0 % 128, LANES)] = flo * v
            c1 = c0 + LANES
            sbuf[j, c1 // 128, pl.ds(c1 % 128, LANES)] = fhi * v
        return 0
      lax.fori_loop(0, CHUNK // LANES, _s16, 0)

    _unpack(0, 0)
    pltpu.async_copy(xi_hbm.at[colv.at[0]], gbufa, sema)

    def _pair(i2, _):
      i = i2 * 2
      _unpack(i + 1, 1)
      pltpu.make_async_copy(xi_hbm.at[colv.at[0]], gbufa, sema).wait()
      pltpu.async_copy(xi_hbm.at[colv.at[1]], gbufb, semb)
      _scale(gbufa, i)
      pltpu.sync_copy(sbuf, acc.at[rowv.at[0]], add=True)

      @pl.when(i + 2 < nchunk)
      def _():
        _unpack(i + 2, 0)

      pltpu.make_async_copy(xi_hbm.at[colv.at[1]], gbufb, semb).wait()

      @pl.when(i + 2 < nchunk)
      def _():
        pltpu.async_copy(xi_hbm.at[colv.at[0]], gbufa, sema)

      _scale(gbufb, i + 1)
      pltpu.sync_copy(sbuf, acc.at[rowv.at[1]], add=True)
      return 0

    lax.fori_loop(0, nchunk // 2, _pair, 0)
    plsc.subcore_barrier()

    pltpu.sync_copy(
        acc.at[pl.ds(sid * rows_per_tile, rows_per_tile)],
        out_hbm.at[cid, pl.ds(sid * rows_per_tile, rows_per_tile)])

  return k


def _combine_body(p_ref, o_ref):
  o_ref[...] = jnp.transpose(p_ref[0] + p_ref[1], (1, 0))


def _combine(partials):
  nblk = OUT_DIM // 256
  return pl.pallas_call(
      _combine_body,
      grid=(nblk,),
      in_specs=[pl.BlockSpec((2, 256, BATCH), lambda i: (0, i, 0))],
      out_specs=pl.BlockSpec((BATCH, 256), lambda i: (0, i)),
      out_shape=jax.ShapeDtypeStruct((BATCH, OUT_DIM), jnp.float32),
  )(partials)


def kernel(x, sparse_values, row, col):
  nnz = sparse_values.shape[0]
  per_tile = -(-nnz // (NW * 2 * CHUNK)) * 2 * CHUNK
  nchunk = per_tile // CHUNK
  ntot = NW * per_tile
  pad = ntot - nnz

  row32 = row.astype(jnp.int32)
  col32 = col.astype(jnp.int32)
  vals = sparse_values
  if pad:
    row32 = jnp.concatenate([row32, jnp.zeros((pad,), jnp.int32)])
    col32 = jnp.concatenate([col32, jnp.zeros((pad,), jnp.int32)])
    vals = jnp.concatenate([vals, jnp.zeros((pad,), jnp.float32)])
  rc = (row32 << 12) | col32
  rc3 = rc.reshape(NW, nchunk, CHUNK)
  val3 = vals.reshape(NW, nchunk, CHUNK)

  xb = x.T.astype(jnp.bfloat16)
  u16 = jax.lax.bitcast_convert_type(xb, jnp.uint16)
  u4 = u16.reshape(IN_DIM, BATCH // 32, 2, 16)
  lo = u4[:, :, 0, :].astype(jnp.uint32)
  hi = u4[:, :, 1, :].astype(jnp.uint32)
  xi = jax.lax.bitcast_convert_type(lo | (hi << 16), jnp.int32)
  xi = xi.reshape(IN_DIM, PKW)

  partials = _sc_spmm(nchunk)(xi, rc3, val3).reshape(2, OUT_DIM, BATCH)
  return _combine(partials)


# speedup vs baseline: 2.0488x; 2.0488x over previous
import functools

import jax
import jax.numpy as jnp
from jax import lax
from jax.experimental import pallas as pl
from jax.experimental.pallas import tpu as pltpu
from jax.experimental.pallas import tpu_sc as plsc

IN_DIM = 4096
OUT_DIM = 4096
BATCH = 256
HALF = BATCH // 2

NUM_TILES = 16
CHUNK = 128
LANES = 16


def _sc_spmm(nchunk):
  mesh = plsc.VectorSubcoreMesh(core_axis_name="c", subcore_axis_name="s")

  @functools.partial(
      pl.kernel,
      mesh=mesh,
      out_type=jax.ShapeDtypeStruct((2, OUT_DIM, HALF), jnp.float32),
      scratch_types=[
          pltpu.VMEM((nchunk, CHUNK), jnp.int32),
          pltpu.VMEM((nchunk, CHUNK), jnp.int32),
          pltpu.VMEM((nchunk, CHUNK), jnp.float32),
          pltpu.VMEM((CHUNK, HALF), jnp.float32),
          pltpu.VMEM((CHUNK, HALF), jnp.float32),
          pltpu.VMEM_SHARED((OUT_DIM, HALF), jnp.float32),
          pltpu.SemaphoreType.DMA,
          pltpu.SemaphoreType.DMA,
      ],
  )
  def k(xs_hbm, row_hbm, col_hbm, val_hbm, out_hbm,
        col_v, row_v, val_v, gbufa, gbufb, acc, sema, semb):
    cid = lax.axis_index("c")
    sid = lax.axis_index("s")

    pltpu.sync_copy(col_hbm.at[sid], col_v)
    pltpu.sync_copy(row_hbm.at[sid], row_v)
    pltpu.sync_copy(val_hbm.at[sid], val_v)

    col_off = cid * IN_DIM

    def _offrow(ch, _):
      def _off(g, _):
        s = pl.ds(g * LANES, LANES)
        col_v[ch, s] = col_v[ch, s] + col_off
        return 0
      lax.fori_loop(0, CHUNK // LANES, _off, 0, unroll=True)
      return 0
    lax.fori_loop(0, nchunk, _offrow, 0)

    def _zrow(i, _):
      def _zlane(g, _):
        gbufa[i, pl.ds(g * LANES, LANES)] = jnp.zeros((LANES,), jnp.float32)
        return 0
      lax.fori_loop(0, HALF // LANES, _zlane, 0, unroll=True)
      return 0
    lax.fori_loop(0, CHUNK, _zrow, 0)
    rows_per_tile = OUT_DIM // NUM_TILES
    pltpu.sync_copy(gbufa, acc.at[pl.ds(sid * rows_per_tile, CHUNK)])
    pltpu.sync_copy(gbufa, acc.at[pl.ds(sid * rows_per_tile + CHUNK, CHUNK)])
    plsc.subcore_barrier()

    def _bcast(vvec, l):
      return lax.gather(
          vvec,
          jnp.full((LANES, 1), l, jnp.int32),
          lax.GatherDimensionNumbers(
              offset_dims=(), collapsed_slice_dims=(0,),
              start_index_map=(0,)),
          (1,),
          mode=lax.GatherScatterMode.PROMISE_IN_BOUNDS)

    def _scale(gbuf, vrow):
      def _s16(j16, _):
        vvec = val_v[vrow, pl.ds(j16 * LANES, LANES)]
        for l in range(LANES):
          v = _bcast(vvec, l)
          j = j16 * LANES + l
          for g in range(HALF // LANES):
            s = pl.ds(g * LANES, LANES)
            gbuf[j, s] = gbuf[j, s] * v
        return 0
      lax.fori_loop(0, CHUNK // LANES, _s16, 0)

    pltpu.async_copy(xs_hbm.at[col_v.at[0]], gbufa, sema)

    def _pair(i2, _):
      i = i2 * 2
      pltpu.make_async_copy(xs_hbm.at[col_v.at[0]], gbufa, sema).wait()
      pltpu.async_copy(xs_hbm.at[col_v.at[i + 1]], gbufb, semb)
      _scale(gbufa, i)
      pltpu.sync_copy(gbufa, acc.at[row_v.at[i]], add=True)

      pltpu.make_async_copy(xs_hbm.at[col_v.at[0]], gbufb, semb).wait()

      @pl.when(i + 2 < nchunk)
      def _():
        pltpu.async_copy(xs_hbm.at[col_v.at[i + 2]], gbufa, sema)

      _scale(gbufb, i + 1)
      pltpu.sync_copy(gbufb, acc.at[row_v.at[i + 1]], add=True)
      return 0

    lax.fori_loop(0, nchunk // 2, _pair, 0)
    plsc.subcore_barrier()

    pltpu.sync_copy(
        acc.at[pl.ds(sid * rows_per_tile, rows_per_tile)],
        out_hbm.at[cid, pl.ds(sid * rows_per_tile, rows_per_tile)])

  return k


def _combine_body(p_ref, o_ref):
  o_ref[...] = jnp.transpose(p_ref[0], (1, 0))


def _combine(partials):
  nblk = OUT_DIM // 256
  return pl.pallas_call(
      _combine_body,
      grid=(2, nblk),
      in_specs=[pl.BlockSpec((1, 256, HALF), lambda c, i: (c, i, 0))],
      out_specs=pl.BlockSpec((HALF, 256), lambda c, i: (c, i)),
      out_shape=jax.ShapeDtypeStruct((BATCH, OUT_DIM), jnp.float32),
  )(partials)


def kernel(x, sparse_values, row, col):
  nnz = sparse_values.shape[0]
  per_tile = -(-nnz // (NUM_TILES * 2 * CHUNK)) * 2 * CHUNK
  nchunk = per_tile // CHUNK
  ntot = NUM_TILES * per_tile
  pad = ntot - nnz

  row32 = row.astype(jnp.int32)
  col32 = col.astype(jnp.int32)
  vals = sparse_values
  if pad:
    row32 = jnp.concatenate([row32, jnp.zeros((pad,), jnp.int32)])
    col32 = jnp.concatenate([col32, jnp.zeros((pad,), jnp.int32)])
    vals = jnp.concatenate([vals, jnp.zeros((pad,), jnp.float32)])
  row3 = row32.reshape(NUM_TILES, nchunk, CHUNK)
  col3 = col32.reshape(NUM_TILES, nchunk, CHUNK)
  val3 = vals.reshape(NUM_TILES, nchunk, CHUNK)

  xs = jnp.transpose(x.reshape(2, HALF, IN_DIM), (0, 2, 1)).reshape(
      2 * IN_DIM, HALF)

  partials = _sc_spmm(nchunk)(xs, row3, col3, val3)
  return _combine(partials)
